# baseline (speedup 1.0000x reference)
import functools

import jax
import jax.numpy as jnp
from jax import lax
from jax.experimental import pallas as pl
from jax.experimental.pallas import tpu as pltpu
from jax.experimental.pallas import tpu_sc as plsc

_K_NN = 5
_W = 32
_L = 16
_LW = 128


def _dist_body(lab_ref, perm_ref, *rest):
    q_refs = rest[:_W]
    queue_refs = rest[_W:2 * _W]
    dist_ref = rest[2 * _W]
    dists = []
    for j in range(_W):
        tile = queue_refs[j][0]
        qv = q_refs[j][0]
        qcol = qv.reshape(tile.shape[0], 1)
        dot = jnp.sum(tile * qcol, axis=0, keepdims=True)
        nrm = jnp.sum(tile * tile, axis=0, keepdims=True)
        inv = 1.0 / (jnp.sqrt(nrm) + 1e-12)
        dists.append(nrm * inv * inv - 2.0 * dot * inv)
    dist_ref[:, 0, 0] = jnp.concatenate(dists, axis=0)


def _tc_dist(lab_s, perm, q, queue, m, d, s):
    rows = m // _W

    def out_map(i, lr, pr):
        return (0, i, 0, 0)

    def q_map(j):
        def f(i, lr, pr):
            return (pr[j * rows + i], 0, 0)
        return f

    def queue_map(j):
        def f(i, lr, pr):
            return (lr[j * rows + i], 0, 0)
        return f

    dist = pl.pallas_call(
        _dist_body,
        grid_spec=pltpu.PrefetchScalarGridSpec(
            num_scalar_prefetch=2,
            grid=(rows,),
            in_specs=[pl.BlockSpec((1, 1, d), q_map(j)) for j in range(_W)] +
                     [pl.BlockSpec((1, d, s), queue_map(j)) for j in range(_W)],
            out_specs=pl.BlockSpec((_W, 1, 1, s), out_map),
        ),
        out_shape=jax.ShapeDtypeStruct((_W, rows, 1, s), jnp.float32),
    )(lab_s, perm, *([q.reshape(-1, 1, d)] * _W), *([queue] * _W))
    return dist.reshape(m, s)


def _bcast16(x, dtype=jnp.int32):
    return jnp.full((_L,), x, dtype)


def _iota16():
    return lax.broadcasted_iota(jnp.int32, (_L,), 0)


def _take16(v, idxvec):
    dnums = lax.GatherDimensionNumbers(
        offset_dims=(), collapsed_slice_dims=(0,), start_index_map=(0,))
    return lax.gather(v, idxvec[:, None], dnums, slice_sizes=(1,),
                      mode=lax.GatherScatterMode.PROMISE_IN_BOUNDS)


def _lanemin16(v):
    for sh in (8, 4, 2, 1):
        v = jnp.minimum(v, _take16(v, _iota16() ^ sh))
    return v


def _make_sc_select(m, s):
    info = plsc.get_sparse_core_info()
    nw = info.num_cores * info.num_subcores
    per_w = m // nw
    sgr = s // _L
    mesh = plsc.VectorSubcoreMesh(core_axis_name="c", subcore_axis_name="s")

    @functools.partial(
        pl.kernel,
        out_type=jax.ShapeDtypeStruct((m,), jnp.int32),
        mesh=mesh,
        scratch_types=[
            pltpu.VMEM((per_w, s), jnp.float32),
            pltpu.VMEM((s,), jnp.float32),
            pltpu.VMEM((per_w,), jnp.int32),
        ],
    )
    def sc_select(dist_hbm, out_hbm, dist_v, row_v, kth_v):
        wid = lax.axis_index("s") * info.num_cores + lax.axis_index("c")
        base = wid * per_w
        pltpu.sync_copy(dist_hbm.at[pl.ds(base, per_w)], dist_v)
        iota = _iota16()
        pos_inf = jnp.float32(jnp.inf)

        def one_group(g, _):
            def one_query(l, acc):
                i = g * _L + l

                def cp_row(k, _):
                    row_v[pl.ds(k * _L, _L)] = dist_v[i, pl.ds(k * _L, _L)]
                    return 0

                lax.fori_loop(0, sgr, cp_row, 0)

                kth_b = _bcast16(0)
                for _r in range(_K_NN):
                    def run_min(k, mv):
                        return jnp.minimum(mv, row_v[pl.ds(k * _L, _L)])

                    mvec = lax.fori_loop(0, sgr, run_min,
                                         jnp.full((_L,), pos_inf, jnp.float32))
                    lo_b = _lanemin16(mvec)

                    def find_min_idx(k, gm):
                        v = row_v[pl.ds(k * _L, _L)]
                        cand = jnp.where(v <= lo_b, k * _L + iota,
                                         jnp.int32(2 ** 30))
                        return jnp.minimum(gm, cand)

                    gmin = lax.fori_loop(0, sgr, find_min_idx,
                                         jnp.full((_L,), 2 ** 30, jnp.int32))
                    kth_b = _lanemin16(gmin)
                    kc = kth_b[0] // _L
                    blk = row_v[pl.ds(kc * _L, _L)]
                    row_v[pl.ds(kc * _L, _L)] = jnp.where(
                        iota == kth_b % _L, pos_inf, blk)

                return jnp.where(iota == l, kth_b, acc)

            picks = lax.fori_loop(0, _L, one_query, jnp.zeros((_L,), jnp.int32))
            kth_v[pl.ds(g * _L, _L)] = picks
            return 0

        lax.fori_loop(0, per_w // _L, one_group, 0)
        pltpu.sync_copy(kth_v, out_hbm.at[pl.ds(base, per_w)])

    return sc_select


def _extract_body(lab_ref, ks_ref, *rest):
    queue_refs = rest[:_W]
    out_ref = rest[_W]
    rows = ks_ref.shape[0] // _W
    i = pl.program_id(0)
    lane = jax.lax.broadcasted_iota(jnp.int32, (1, _LW), 1)
    for j in range(_W):
        win = queue_refs[j][0]
        kth = ks_ref[j * rows + i]
        mask = (lane == kth % _LW).astype(jnp.float32)
        col = jax.lax.dot_general(
            mask, win, (((1,), (1,)), ((), ())),
            preferred_element_type=jnp.float32)
        nrm = jnp.sum(col * col)
        out_ref[j, 0] = col / (jnp.sqrt(nrm) + 1e-12)


def _tc_extract(lab_s, ks, queue, m, d, s):
    rows = m // _W

    def out_map(i, lr, kr):
        return (0, i, 0, 0)

    def queue_map(j):
        def f(i, lr, kr):
            return (lr[j * rows + i], 0, kr[j * rows + i] // _LW)
        return f

    rep = pl.pallas_call(
        _extract_body,
        grid_spec=pltpu.PrefetchScalarGridSpec(
            num_scalar_prefetch=2,
            grid=(rows,),
            in_specs=[pl.BlockSpec((1, d, _LW), queue_map(j))
                      for j in range(_W)],
            out_specs=pl.BlockSpec((_W, 1, 1, d), out_map),
        ),
        out_shape=jax.ShapeDtypeStruct((_W, rows, 1, d), jnp.float32),
    )(lab_s, ks, *([queue] * _W))
    return rep.reshape(m, d)


def kernel(q, labels, queue):
    n, d = q.shape
    c, _, s = queue.shape
    m = n // 2
    lab = labels[:m].astype(jnp.int32)
    packed = jnp.sort(lab * 2048 + jnp.arange(m, dtype=jnp.int32))
    perm = packed & 2047
    lab_s = packed >> 11
    sel = _make_sc_select(m, s)
    dist = _tc_dist(lab_s, perm, q, queue, m, d, s)
    ks = sel(dist)
    rep = _tc_extract(lab_s, ks, queue, m, d, s)
    return q.at[perm].set(rep)


# speedup vs baseline: 1.0851x; 1.0851x over previous
import functools

import jax
import jax.numpy as jnp
from jax import lax
from jax.experimental import pallas as pl
from jax.experimental.pallas import tpu as pltpu
from jax.experimental.pallas import tpu_sc as plsc

_K_NN = 5
_W = 32
_L = 16
_LW = 128


def _dist_body(lab_ref, perm_ref, *rest):
    q_refs = rest[:_W]
    queue_refs = rest[_W:2 * _W]
    dist_ref = rest[2 * _W]
    dists = []
    for j in range(_W):
        tile = queue_refs[j][0]
        qv = q_refs[j][0]
        qcol = qv.reshape(tile.shape[0], 1)
        dot = jnp.sum(tile * qcol, axis=0, keepdims=True)
        nrm = jnp.sum(tile * tile, axis=0, keepdims=True)
        inv = 1.0 / (jnp.sqrt(nrm) + 1e-12)
        dists.append(nrm * inv * inv - 2.0 * dot * inv)
    dist_ref[:, 0, 0] = jnp.concatenate(dists, axis=0)


def _tc_dist(lab_s, perm, q, queue, m, d, s):
    rows = m // _W

    def out_map(i, lr, pr):
        return (0, i, 0, 0)

    def q_map(j):
        def f(i, lr, pr):
            return (pr[j * rows + i], 0, 0)
        return f

    def queue_map(j):
        def f(i, lr, pr):
            return (lr[j * rows + i], 0, 0)
        return f

    dist = pl.pallas_call(
        _dist_body,
        grid_spec=pltpu.PrefetchScalarGridSpec(
            num_scalar_prefetch=2,
            grid=(rows,),
            in_specs=[pl.BlockSpec((1, 1, d), q_map(j)) for j in range(_W)] +
                     [pl.BlockSpec((1, d, s), queue_map(j)) for j in range(_W)],
            out_specs=pl.BlockSpec((_W, 1, 1, s), out_map),
        ),
        out_shape=jax.ShapeDtypeStruct((_W, rows, 1, s), jnp.float32),
    )(lab_s, perm, *([q.reshape(-1, 1, d)] * _W), *([queue] * _W))
    return dist.reshape(m, s)


def _bcast16(x, dtype=jnp.int32):
    return jnp.full((_L,), x, dtype)


def _iota16():
    return lax.broadcasted_iota(jnp.int32, (_L,), 0)


def _take16(v, idxvec):
    dnums = lax.GatherDimensionNumbers(
        offset_dims=(), collapsed_slice_dims=(0,), start_index_map=(0,))
    return lax.gather(v, idxvec[:, None], dnums, slice_sizes=(1,),
                      mode=lax.GatherScatterMode.PROMISE_IN_BOUNDS)


def _lanemin16(v):
    for sh in (8, 4, 2, 1):
        v = jnp.minimum(v, _take16(v, _iota16() ^ sh))
    return v


def _make_sc_select(m, s):
    info = plsc.get_sparse_core_info()
    nw = info.num_cores * info.num_subcores
    per_w = m // nw
    sgr = s // _L
    mesh = plsc.VectorSubcoreMesh(core_axis_name="c", subcore_axis_name="s")

    @functools.partial(
        pl.kernel,
        out_type=jax.ShapeDtypeStruct((m,), jnp.int32),
        mesh=mesh,
        scratch_types=[
            pltpu.VMEM((per_w, s), jnp.float32),
            pltpu.VMEM((s,), jnp.float32),
            pltpu.VMEM((per_w,), jnp.int32),
        ],
    )
    def sc_select(dist_hbm, out_hbm, dist_v, row_v, kth_v):
        wid = lax.axis_index("s") * info.num_cores + lax.axis_index("c")
        base = wid * per_w
        pltpu.sync_copy(dist_hbm.at[pl.ds(base, per_w)], dist_v)
        iota = _iota16()
        pos_inf = jnp.float32(jnp.inf)

        def one_group(g, _):
            def one_query(l, acc):
                i = g * _L + l

                def cp_row(k, _):
                    row_v[pl.ds(k * _L, _L)] = dist_v[i, pl.ds(k * _L, _L)]
                    return 0

                lax.fori_loop(0, sgr, cp_row, 0)

                kth_b = _bcast16(0)
                for _r in range(_K_NN):
                    def run_min(k, mv):
                        return jnp.minimum(mv, row_v[pl.ds(k * _L, _L)])

                    mvec = lax.fori_loop(0, sgr, run_min,
                                         jnp.full((_L,), pos_inf, jnp.float32))
                    lo_b = _lanemin16(mvec)

                    def find_min_idx(k, gm):
                        v = row_v[pl.ds(k * _L, _L)]
                        cand = jnp.where(v <= lo_b, k * _L + iota,
                                         jnp.int32(2 ** 30))
                        return jnp.minimum(gm, cand)

                    gmin = lax.fori_loop(0, sgr, find_min_idx,
                                         jnp.full((_L,), 2 ** 30, jnp.int32))
                    kth_b = _lanemin16(gmin)
                    kc = kth_b[0] // _L
                    blk = row_v[pl.ds(kc * _L, _L)]
                    row_v[pl.ds(kc * _L, _L)] = jnp.where(
                        iota == kth_b % _L, pos_inf, blk)

                return jnp.where(iota == l, kth_b, acc)

            picks = lax.fori_loop(0, _L, one_query, jnp.zeros((_L,), jnp.int32))
            kth_v[pl.ds(g * _L, _L)] = picks
            return 0

        lax.fori_loop(0, per_w // _L, one_group, 0)
        pltpu.sync_copy(kth_v, out_hbm.at[pl.ds(base, per_w)])

    return sc_select


def _extract_body(lab_ref, ks_ref, *rest):
    queue_refs = rest[:_W]
    out_ref = rest[_W]
    rows = ks_ref.shape[0] // _W
    i = pl.program_id(0)
    lane = jax.lax.broadcasted_iota(jnp.int32, (1, _LW), 1)
    for j in range(_W):
        win = queue_refs[j][0]
        kth = ks_ref[j * rows + i]
        mask = (lane == kth % _LW).astype(jnp.float32)
        col = jax.lax.dot_general(
            mask, win, (((1,), (1,)), ((), ())),
            preferred_element_type=jnp.float32)
        nrm = jnp.sum(col * col)
        out_ref[j, 0] = col / (jnp.sqrt(nrm) + 1e-12)


def _tc_extract(lab_s, ks, queue, m, d, s):
    rows = m // _W

    def out_map(i, lr, kr):
        return (0, i, 0, 0)

    def queue_map(j):
        def f(i, lr, kr):
            return (lr[j * rows + i], 0, kr[j * rows + i] // _LW)
        return f

    rep = pl.pallas_call(
        _extract_body,
        grid_spec=pltpu.PrefetchScalarGridSpec(
            num_scalar_prefetch=2,
            grid=(rows,),
            in_specs=[pl.BlockSpec((1, d, _LW), queue_map(j))
                      for j in range(_W)],
            out_specs=pl.BlockSpec((_W, 1, 1, d), out_map),
        ),
        out_shape=jax.ShapeDtypeStruct((_W, rows, 1, d), jnp.float32),
    )(lab_s, ks, *([queue] * _W))
    return rep.reshape(m, d)


def kernel(q, labels, queue):
    n, d = q.shape
    c, _, s = queue.shape
    m = n // 2
    lab = labels[:m].astype(jnp.int32)
    packed = jnp.sort(lab * 2048 + jnp.arange(m, dtype=jnp.int32))
    perm = packed & 2047
    lab_s = packed >> 11
    mh = m // 2
    sel = _make_sc_select(mh, s)
    dist1 = _tc_dist(lab_s[:mh], perm[:mh], q, queue, mh, d, s)
    ks1 = sel(dist1)
    dist2 = _tc_dist(lab_s[mh:], perm[mh:], q, queue, mh, d, s)
    ks2 = sel(dist2)
    rep1 = _tc_extract(lab_s[:mh], ks1, queue, mh, d, s)
    rep2 = _tc_extract(lab_s[mh:], ks2, queue, mh, d, s)
    return q.at[perm].set(jnp.concatenate([rep1, rep2]))
